# baseline (speedup 1.0000x reference)
import functools

import jax
import jax.numpy as jnp
from jax import lax
from jax.experimental import pallas as pl
from jax.experimental.pallas import tpu as pltpu
from jax.experimental.pallas import tpu_sc as plsc

GX = GY = GZ = 128
C = 8
NC, NS, L = 2, 16, 16
NW = NC * NS
CHUNK = 512
GROUPS = CHUNK // L
NROW = 8 * CHUNK
IDXR = NROW // 128


def _build(np_total):
    per_w = np_total // NW
    n_chunks = per_w // CHUNK
    mesh = plsc.VectorSubcoreMesh(core_axis_name="c", subcore_axis_name="s")

    @functools.partial(
        pl.kernel,
        mesh=mesh,
        out_type=jax.ShapeDtypeStruct((np_total * C,), jnp.float32),
        scratch_types=[
            pltpu.VMEM((16,), jnp.float32),
            pltpu.VMEM((CHUNK * 3,), jnp.float32),
            pltpu.VMEM((CHUNK,), jnp.float32),
            pltpu.VMEM((CHUNK,), jnp.float32),
            pltpu.VMEM((CHUNK,), jnp.float32),
            pltpu.VMEM((NROW,), jnp.int32),
            pltpu.VMEM((NROW, C), jnp.float32),
            pltpu.VMEM((CHUNK * C,), jnp.float32),
            pltpu.SemaphoreType.DMA,
        ],
        compiler_params=pltpu.CompilerParams(
            needs_layout_passes=False, use_tc_tiling_on_sc=False),
    )
    def grid_lookup(coords_hbm, table_hbm, params_hbm, out_hbm,
                    params_v, coords_v, wxb, wyb, wzb, idxb, gbuf, outb, sem):
        wid = lax.axis_index("s") * NC + lax.axis_index("c")
        base_w = wid * per_w
        pltpu.sync_copy(params_hbm, params_v)
        lanes = jnp.arange(L, dtype=jnp.int32)
        lanes3 = lanes * 3
        pv = params_v[...]
        lo0 = pv[0]
        lo1 = pv[1]
        lo2 = pv[2]
        iv0 = pv[3]
        iv1 = pv[4]
        iv2 = pv[5]

        def chunk_body(t, carry):
            b = base_w + t * CHUNK
            pltpu.sync_copy(coords_hbm.at[pl.ds(b * 3, CHUNK * 3)], coords_v)

            @plsc.parallel_loop(0, GROUPS, unroll=4)
            def idx_body(g):
                s = g * L
                p3 = s * 3 + lanes3
                x = plsc.load_gather(coords_v, [p3])
                y = plsc.load_gather(coords_v, [p3 + 1])
                z = plsc.load_gather(coords_v, [p3 + 2])
                px = jnp.clip((x - lo0) * iv0, 0.0, GX - 1.0)
                py = jnp.clip((y - lo1) * iv1, 0.0, GY - 1.0)
                pz = jnp.clip((z - lo2) * iv2, 0.0, GZ - 1.0)
                ix0 = px.astype(jnp.int32)
                iy0 = py.astype(jnp.int32)
                iz0 = pz.astype(jnp.int32)
                wxb[pl.ds(s, L)] = px - ix0.astype(jnp.float32)
                wyb[pl.ds(s, L)] = py - iy0.astype(jnp.float32)
                wzb[pl.ds(s, L)] = pz - iz0.astype(jnp.float32)
                iz1 = jnp.minimum(iz0 + 1, GZ - 1)
                xs0 = ix0 * (GY * GZ)
                xs1 = jnp.minimum(ix0 + 1, GX - 1) * (GY * GZ)
                ys0 = iy0 * GZ
                ys1 = jnp.minimum(iy0 + 1, GY - 1) * GZ
                for j in range(8):
                    xs = xs1 if (j >> 2) & 1 else xs0
                    ys = ys1 if (j >> 1) & 1 else ys0
                    zs = iz1 if j & 1 else iz0
                    idxb[pl.ds(j * CHUNK + s, L)] = xs + ys + zs

            copies = [
                pltpu.async_copy(table_hbm.at[idxb.at[pl.ds(r * 128, 128)]],
                                 gbuf.at[pl.ds(r * 128, 128)], sem)
                for r in range(IDXR)
            ]
            for cp in copies:
                cp.wait()

            @plsc.parallel_loop(0, GROUPS, unroll=2)
            def blend_body(g):
                s = g * L
                pid = s + lanes
                wxv = wxb[pl.ds(s, L)]
                wyv = wyb[pl.ds(s, L)]
                wzv = wzb[pl.ds(s, L)]
                ux = 1.0 - wxv
                uy = 1.0 - wyv
                uz = 1.0 - wzv
                a00 = ux * uy
                a01 = ux * wyv
                a10 = wxv * uy
                a11 = wxv * wyv
                w8 = [a00 * uz, a00 * wzv, a01 * uz, a01 * wzv,
                      a10 * uz, a10 * wzv, a11 * uz, a11 * wzv]
                for ch in range(C):
                    cc = jnp.full((L,), ch, jnp.int32)
                    acc = w8[0] * plsc.load_gather(gbuf, [pid, cc])
                    for j in range(1, 8):
                        acc = acc + w8[j] * plsc.load_gather(
                            gbuf, [j * CHUNK + pid, cc])
                    plsc.store_scatter(outb, [pid * 8 + ch], acc)

            pltpu.sync_copy(outb, out_hbm.at[pl.ds(b * C, CHUNK * C)])
            return carry

        lax.fori_loop(0, n_chunks, chunk_body, 0)

    return grid_lookup


@jax.jit
def kernel(coords, grid, bbox_min, bbox_max):
    n = coords.shape[0]
    coords = coords[:, :3]
    np_total = NW * CHUNK * pl.cdiv(n, NW * CHUNK)
    pad = np_total - n
    f = (jnp.arange(pad, dtype=jnp.float32) + 0.5) / max(pad, 1)
    filler = jnp.stack([f, jnp.mod(f * 7.0, 1.0), jnp.mod(f * 13.0, 1.0)], axis=1)
    scale = jnp.clip(bbox_max - bbox_min, 1e-6, None)
    coords_p = jnp.concatenate([coords, filler * scale + bbox_min], axis=0)
    params = jnp.concatenate(
        [bbox_min.astype(jnp.float32), (GX - 1.0) / scale,
         jnp.zeros((10,), jnp.float32)])
    table = grid.reshape(-1, C)
    out = _build(np_total)(coords_p.reshape(-1), table, params)
    return out.reshape(np_total, C)[:n]


# speedup vs baseline: 1.9645x; 1.9645x over previous
import functools

import jax
import jax.numpy as jnp
from jax import lax
from jax.experimental import pallas as pl
from jax.experimental.pallas import tpu as pltpu
from jax.experimental.pallas import tpu_sc as plsc

GX = GY = GZ = 128
C = 8
NC, NS, L = 2, 16, 16
NW = NC * NS
CHUNK = 512
GROUPS = CHUNK // L
NROW = 8 * CHUNK
IDXR = NROW // 128


def _build(np_total):
    per_w = np_total // NW
    n_chunks = per_w // CHUNK
    mesh = plsc.VectorSubcoreMesh(core_axis_name="c", subcore_axis_name="s")

    @functools.partial(
        pl.kernel,
        mesh=mesh,
        out_type=jax.ShapeDtypeStruct((np_total * C,), jnp.float32),
        scratch_types=[
            pltpu.VMEM((16,), jnp.float32),
            pltpu.VMEM((CHUNK * 3,), jnp.float32),
            pltpu.VMEM((2 * CHUNK,), jnp.float32),
            pltpu.VMEM((2 * CHUNK,), jnp.float32),
            pltpu.VMEM((2 * CHUNK,), jnp.float32),
            pltpu.VMEM((2 * NROW,), jnp.int32),
            pltpu.VMEM((2 * NROW, C), jnp.float32),
            pltpu.VMEM((CHUNK * C,), jnp.float32),
            pltpu.SemaphoreType.DMA,
        ],
        compiler_params=pltpu.CompilerParams(
            needs_layout_passes=False, use_tc_tiling_on_sc=False),
    )
    def grid_lookup(coords_hbm, table_hbm, params_hbm, out_hbm,
                    params_v, coords_v, wxb, wyb, wzb, idxb, gbuf, outb, sem):
        wid = lax.axis_index("s") * NC + lax.axis_index("c")
        base_w = wid * per_w
        pltpu.sync_copy(params_hbm, params_v)
        lanes = jnp.arange(L, dtype=jnp.int32)
        pv = params_v[...]
        lo0 = pv[0]
        lo1 = pv[1]
        lo2 = pv[2]
        iv0 = pv[3]
        iv1 = pv[4]
        iv2 = pv[5]

        def stage(t, buf):
            b = base_w + t * CHUNK
            po = buf * CHUNK
            pio = buf * NROW
            pltpu.sync_copy(coords_hbm.at[pl.ds(b * 3, CHUNK * 3)], coords_v)

            @plsc.parallel_loop(0, GROUPS, unroll=4)
            def idx_body(g):
                s = g * L
                x = coords_v[pl.ds(s, L)]
                y = coords_v[pl.ds(CHUNK + s, L)]
                z = coords_v[pl.ds(2 * CHUNK + s, L)]
                px = jnp.clip((x - lo0) * iv0, 0.0, GX - 1.0)
                py = jnp.clip((y - lo1) * iv1, 0.0, GY - 1.0)
                pz = jnp.clip((z - lo2) * iv2, 0.0, GZ - 1.0)
                ix0 = px.astype(jnp.int32)
                iy0 = py.astype(jnp.int32)
                iz0 = pz.astype(jnp.int32)
                wxb[pl.ds(po + s, L)] = px - ix0.astype(jnp.float32)
                wyb[pl.ds(po + s, L)] = py - iy0.astype(jnp.float32)
                wzb[pl.ds(po + s, L)] = pz - iz0.astype(jnp.float32)
                iz1 = jnp.minimum(iz0 + 1, GZ - 1)
                xs0 = ix0 * (GY * GZ)
                xs1 = jnp.minimum(ix0 + 1, GX - 1) * (GY * GZ)
                ys0 = iy0 * GZ
                ys1 = jnp.minimum(iy0 + 1, GY - 1) * GZ
                for j in range(8):
                    xs = xs1 if (j >> 2) & 1 else xs0
                    ys = ys1 if (j >> 1) & 1 else ys0
                    zs = iz1 if j & 1 else iz0
                    idxb[pl.ds(pio + j * CHUNK + s, L)] = xs + ys + zs

        def fire(buf):
            pio = buf * NROW
            for r in range(IDXR):
                pltpu.async_copy(
                    table_hbm.at[idxb.at[pl.ds(pio + r * 128, 128)]],
                    gbuf.at[pl.ds(pio + r * 128, 128)], sem)

        def drain(buf):
            pio = buf * NROW
            for r in range(IDXR):
                pltpu.make_async_copy(
                    table_hbm.at[idxb.at[pl.ds(pio + r * 128, 128)]],
                    gbuf.at[pl.ds(pio + r * 128, 128)], sem).wait()

        def blend(t, buf):
            b = base_w + t * CHUNK
            po = buf * CHUNK
            pio = buf * NROW

            @plsc.parallel_loop(0, GROUPS, unroll=4)
            def blend_body(g):
                s = g * L
                pid = s + lanes
                wxv = wxb[pl.ds(po + s, L)]
                wyv = wyb[pl.ds(po + s, L)]
                wzv = wzb[pl.ds(po + s, L)]
                ux = 1.0 - wxv
                uy = 1.0 - wyv
                uz = 1.0 - wzv
                a00 = ux * uy
                a01 = ux * wyv
                a10 = wxv * uy
                a11 = wxv * wyv
                w8 = [a00 * uz, a00 * wzv, a01 * uz, a01 * wzv,
                      a10 * uz, a10 * wzv, a11 * uz, a11 * wzv]
                r0 = pio + pid
                for ch in range(C):
                    cc = jnp.full((L,), ch, jnp.int32)
                    acc = w8[0] * plsc.load_gather(gbuf, [r0, cc])
                    for j in range(1, 8):
                        acc = acc + w8[j] * plsc.load_gather(
                            gbuf, [r0 + j * CHUNK, cc])
                    plsc.store_scatter(outb, [pid * 8 + ch], acc)

            pltpu.sync_copy(outb, out_hbm.at[pl.ds(b * C, CHUNK * C)])

        stage(0, 0)
        fire(0)

        def chunk_body(t, carry):
            par = t % 2
            nxt = 1 - par
            stage(t + 1, nxt)
            drain(par)
            fire(nxt)
            blend(t, par)
            return carry

        lax.fori_loop(0, n_chunks - 1, chunk_body, 0)
        last = n_chunks - 1
        drain(last % 2)
        blend(last, last % 2)

    return grid_lookup


@jax.jit
def kernel(coords, grid, bbox_min, bbox_max):
    n = coords.shape[0]
    coords = coords[:, :3]
    np_total = NW * CHUNK * pl.cdiv(n, NW * CHUNK)
    pad = np_total - n
    f = (jnp.arange(pad, dtype=jnp.float32) + 0.5) / max(pad, 1)
    filler = jnp.stack([f, jnp.mod(f * 7.0, 1.0), jnp.mod(f * 13.0, 1.0)], axis=1)
    scale = jnp.clip(bbox_max - bbox_min, 1e-6, None)
    coords_p = jnp.concatenate([coords, filler * scale + bbox_min], axis=0)
    params = jnp.concatenate(
        [bbox_min.astype(jnp.float32), (GX - 1.0) / scale,
         jnp.zeros((10,), jnp.float32)])
    table = grid.reshape(-1, C)
    out = _build(np_total)(coords_p.T.reshape(-1), table, params)
    return out.reshape(np_total, C)[:n]
